# baseline (speedup 1.0000x reference)
import functools

import jax
import jax.numpy as jnp
from jax.experimental import pallas as pl
from jax.experimental.pallas import tpu as pltpu


NB = 4


def _conv_stack_kernel(xp_ref, b1_ref, bias1_ref, w2_ref, b2_ref, w3_ref,
                       b3_ref, o_ref):
    f32 = jnp.float32
    bf16 = jnp.bfloat16

    x = xp_ref[...].reshape(NB, 73, 2, 146)
    xe = x[:, :, 0, :]
    xo = x[:, :, 1, :]
    acc1 = [[jnp.zeros((NB * 72, 2304), f32) for _ in range(2)]
            for _ in range(2)]
    for hp in range(2):
        for dy in range(3):
            s = hp + dy
            src = xe if s % 2 == 0 else xo
            lhs = src[:, s // 2:s // 2 + 72, :].reshape(NB * 72, 146)
            for wp in range(2):
                acc1[hp][wp] = acc1[hp][wp] + jnp.dot(
                    lhs, b1_ref[wp, dy], preferred_element_type=f32)
    m = jnp.maximum(jnp.maximum(acc1[0][0], acc1[0][1]),
                    jnp.maximum(acc1[1][0], acc1[1][1]))
    y = jnp.maximum(m + bias1_ref[0], 0.0).astype(bf16)
    y = y.reshape(NB, 72, 72, 32)

    zc = jnp.zeros((NB, 72, 1, 32), bf16)
    zr = jnp.zeros((NB, 1, 74, 32), bf16)
    p = jnp.concatenate([zc, y, zc], axis=2)
    p = jnp.concatenate([zr, p, zr], axis=1)
    st = jnp.concatenate([p[:, 0:72], p[:, 1:73], p[:, 2:74]], axis=3)
    stv = st.reshape(NB, 36, 2, 37, 2, 96)
    acc2 = [[jnp.zeros((NB * 36 * 36, 64), f32) for _ in range(2)]
            for _ in range(2)]
    for hp in range(2):
        for wp in range(2):
            for dx in range(3):
                w0 = wp + dx
                patch = stv[:, :, hp, w0 // 2:w0 // 2 + 36, w0 % 2, :]
                patch = patch.reshape(NB * 36 * 36, 96)
                acc2[hp][wp] = acc2[hp][wp] + jnp.dot(
                    patch, w2_ref[dx], preferred_element_type=f32)
    m = jnp.maximum(jnp.maximum(acc2[0][0], acc2[0][1]),
                    jnp.maximum(acc2[1][0], acc2[1][1]))
    y = jnp.maximum(m + b2_ref[0], 0.0).astype(bf16)
    y = y.reshape(NB, 36, 36, 64)

    zc = jnp.zeros((NB, 36, 1, 64), bf16)
    zr = jnp.zeros((NB, 1, 38, 64), bf16)
    p = jnp.concatenate([zc, y, zc], axis=2)
    p = jnp.concatenate([zr, p, zr], axis=1)
    st = jnp.concatenate([p[:, 0:36], p[:, 1:37], p[:, 2:38]], axis=3)
    stv = st.reshape(NB, 18, 2, 19, 2, 192)
    acc3 = [[jnp.zeros((NB * 18 * 18, 128), f32) for _ in range(2)]
            for _ in range(2)]
    for hp in range(2):
        for wp in range(2):
            for dx in range(3):
                w0 = wp + dx
                patch = stv[:, :, hp, w0 // 2:w0 // 2 + 18, w0 % 2, :]
                patch = patch.reshape(NB * 18 * 18, 192)
                acc3[hp][wp] = acc3[hp][wp] + jnp.dot(
                    patch, w3_ref[dx], preferred_element_type=f32)
    m = jnp.maximum(jnp.maximum(acc3[0][0], acc3[0][1]),
                    jnp.maximum(acc3[1][0], acc3[1][1]))
    y = jnp.maximum(m + b3_ref[0], 0.0).astype(bf16)
    o_ref[...] = y.reshape(NB, 18, 18 * 128)


def _fc_kernel(a_ref, w1_ref, b1_ref, w2_ref, o_ref, acc_ref):
    k = pl.program_id(1)

    @pl.when(k == 0)
    def _():
        acc_ref[...] = jnp.zeros_like(acc_ref)

    acc_ref[...] += jnp.dot(a_ref[...], w1_ref[...],
                            preferred_element_type=jnp.float32)

    @pl.when(k == pl.num_programs(1) - 1)
    def _():
        h = jnp.maximum(acc_ref[...] + b1_ref[...], 0.0).astype(jnp.bfloat16)
        o_ref[0] = jnp.dot(h, w2_ref[...],
                           preferred_element_type=jnp.float32)


def kernel(conv1_w, conv1_b, conv2_w, conv2_b, conv3_w, conv3_b,
           fc1_w, fc1_b, fc2_w, fc2_b, x):
    N = x.shape[0]

    xp = jnp.pad(x.reshape(N, 144, 144), ((0, 0), (1, 1), (1, 1)))
    xp = xp.astype(jnp.bfloat16)

    eye = [jnp.eye(146, 144, -dx, dtype=jnp.float32) for dx in range(3)]
    band = sum(eye[dx][None, :, :, None] * conv1_w[:, dx, 0, :][:, None, None, :]
               for dx in range(3))
    band = band.reshape(3, 146, 72, 2, 32)
    band = jnp.moveaxis(band, 3, 0).reshape(2, 3, 146, 72 * 32)
    band = band.astype(jnp.bfloat16)
    bias1 = jnp.tile(conv1_b, 72).reshape(1, 72 * 32)

    w2s = jnp.transpose(conv2_w, (1, 0, 2, 3)).reshape(3, 96, 64)
    w2s = w2s.astype(jnp.bfloat16)
    w3s = jnp.transpose(conv3_w, (1, 0, 2, 3)).reshape(3, 192, 128)
    w3s = w3s.astype(jnp.bfloat16)

    flat = pl.pallas_call(
        _conv_stack_kernel,
        out_shape=jax.ShapeDtypeStruct((N, 18, 18 * 128), jnp.bfloat16),
        grid=(N // NB,),
        in_specs=[
            pl.BlockSpec((NB, 146, 146), lambda i: (i, 0, 0)),
            pl.BlockSpec((2, 3, 146, 2304), lambda i: (0, 0, 0, 0)),
            pl.BlockSpec((1, 2304), lambda i: (0, 0)),
            pl.BlockSpec((3, 96, 64), lambda i: (0, 0, 0)),
            pl.BlockSpec((1, 64), lambda i: (0, 0)),
            pl.BlockSpec((3, 192, 128), lambda i: (0, 0, 0)),
            pl.BlockSpec((1, 128), lambda i: (0, 0)),
        ],
        out_specs=pl.BlockSpec((NB, 18, 18 * 128), lambda i: (i, 0, 0)),
        compiler_params=pltpu.CompilerParams(
            dimension_semantics=("parallel",)),
    )(xp, band, bias1, w2s, conv2_b.reshape(1, 64), w3s,
      conv3_b.reshape(1, 128))

    a_flat = flat.reshape(N, 41472)

    tk = 41472 // 4
    partials = pl.pallas_call(
        _fc_kernel,
        out_shape=jax.ShapeDtypeStruct((2, N, 3), jnp.float32),
        grid=(2, 4),
        in_specs=[
            pl.BlockSpec((N, tk), lambda h, k: (0, k)),
            pl.BlockSpec((tk, 128), lambda h, k: (k, h)),
            pl.BlockSpec((1, 128), lambda h, k: (0, h)),
            pl.BlockSpec((128, 3), lambda h, k: (h, 0)),
        ],
        out_specs=pl.BlockSpec((1, N, 3), lambda h, k: (h, 0, 0)),
        scratch_shapes=[pltpu.VMEM((N, 128), jnp.float32)],
        compiler_params=pltpu.CompilerParams(
            dimension_semantics=("parallel", "arbitrary")),
    )(a_flat, fc1_w, fc1_b.reshape(1, 256).astype(jnp.float32), fc2_w)

    return jnp.sum(partials, axis=0) + fc2_b.reshape(1, 3)


# speedup vs baseline: 1.2109x; 1.2109x over previous
import functools

import jax
import jax.numpy as jnp
from jax.experimental import pallas as pl
from jax.experimental.pallas import tpu as pltpu


NB = 2


def _conv_stack_kernel(xe_ref, xo_ref, b1_ref, bias1_ref, w2_ref, b2_ref,
                       w3_ref, b3_ref, o_ref):
    f32 = jnp.float32
    bf16 = jnp.bfloat16

    xe = xe_ref[...]
    xo = xo_ref[...]
    acc1 = [[None, None], [None, None]]
    for hp in range(2):
        for dy in range(3):
            s = hp + dy
            src = xe if s % 2 == 0 else xo
            lhs = src[:, s // 2:s // 2 + 72, :].reshape(NB * 72, 146)
            for wp in range(2):
                d = jnp.dot(lhs, b1_ref[wp, dy], preferred_element_type=f32)
                acc1[hp][wp] = d if acc1[hp][wp] is None else acc1[hp][wp] + d
    m = jnp.maximum(jnp.maximum(acc1[0][0], acc1[0][1]),
                    jnp.maximum(acc1[1][0], acc1[1][1]))
    y = jnp.maximum(m + bias1_ref[0], 0.0).astype(bf16)
    y = y.reshape(NB, 72, 72, 32)

    zc = jnp.zeros((NB, 72, 1, 32), bf16)
    zr = jnp.zeros((NB, 1, 74, 32), bf16)
    p = jnp.concatenate([zc, y, zc], axis=2)
    p = jnp.concatenate([zr, p, zr], axis=1)
    st = jnp.concatenate([p[:, 0:72], p[:, 1:73], p[:, 2:74]], axis=3)
    stv = st.reshape(NB, 36, 2, 37, 2, 96)
    acc2 = [[None, None], [None, None]]
    for hp in range(2):
        for wp in range(2):
            for dx in range(3):
                w0 = wp + dx
                patch = stv[:, :, hp, w0 // 2:w0 // 2 + 36, w0 % 2, :]
                patch = patch.reshape(NB * 36 * 36, 96)
                d = jnp.dot(patch, w2_ref[dx], preferred_element_type=f32)
                acc2[hp][wp] = d if acc2[hp][wp] is None else acc2[hp][wp] + d
    m = jnp.maximum(jnp.maximum(acc2[0][0], acc2[0][1]),
                    jnp.maximum(acc2[1][0], acc2[1][1]))
    y = jnp.maximum(m + b2_ref[0], 0.0).astype(bf16)
    y = y.reshape(NB, 36, 36, 64)

    zc = jnp.zeros((NB, 36, 1, 64), bf16)
    zr = jnp.zeros((NB, 1, 38, 64), bf16)
    p = jnp.concatenate([zc, y, zc], axis=2)
    p = jnp.concatenate([zr, p, zr], axis=1)
    st = jnp.concatenate([p[:, 0:36], p[:, 1:37], p[:, 2:38]], axis=3)
    stv = st.reshape(NB, 18, 2, 19, 2, 192)
    acc3 = [[None, None], [None, None]]
    for hp in range(2):
        for wp in range(2):
            for dx in range(3):
                w0 = wp + dx
                patch = stv[:, :, hp, w0 // 2:w0 // 2 + 18, w0 % 2, :]
                patch = patch.reshape(NB * 18 * 18, 192)
                d = jnp.dot(patch, w3_ref[dx], preferred_element_type=f32)
                acc3[hp][wp] = d if acc3[hp][wp] is None else acc3[hp][wp] + d
    m = jnp.maximum(jnp.maximum(acc3[0][0], acc3[0][1]),
                    jnp.maximum(acc3[1][0], acc3[1][1]))
    y = jnp.maximum(m + b3_ref[0], 0.0).astype(bf16)
    o_ref[...] = y.reshape(NB, 18, 18 * 128)


def _fc_kernel(a_ref, w1_ref, b1_ref, w2_ref, o_ref, acc_ref):
    k = pl.program_id(1)

    @pl.when(k == 0)
    def _():
        acc_ref[...] = jnp.zeros_like(acc_ref)

    acc_ref[...] += jnp.dot(a_ref[...], w1_ref[...],
                            preferred_element_type=jnp.float32)

    @pl.when(k == pl.num_programs(1) - 1)
    def _():
        h = jnp.maximum(acc_ref[...] + b1_ref[...], 0.0).astype(jnp.bfloat16)
        o_ref[0] = jnp.dot(h, w2_ref[...],
                           preferred_element_type=jnp.float32)


def kernel(conv1_w, conv1_b, conv2_w, conv2_b, conv3_w, conv3_b,
           fc1_w, fc1_b, fc2_w, fc2_b, x):
    N = x.shape[0]

    xp = jnp.pad(x.reshape(N, 144, 144), ((0, 0), (1, 1), (1, 1)))
    xp = xp.astype(jnp.bfloat16)
    xe = xp[:, 0::2, :]
    xo = xp[:, 1::2, :]

    eye = [jnp.eye(146, 144, -dx, dtype=jnp.float32) for dx in range(3)]
    band = sum(eye[dx][None, :, :, None] * conv1_w[:, dx, 0, :][:, None, None, :]
               for dx in range(3))
    band = band.reshape(3, 146, 72, 2, 32)
    band = jnp.moveaxis(band, 3, 0).reshape(2, 3, 146, 72 * 32)
    band = band.astype(jnp.bfloat16)
    bias1 = jnp.tile(conv1_b, 72).reshape(1, 72 * 32)

    w2s = jnp.transpose(conv2_w, (1, 0, 2, 3)).reshape(3, 96, 64)
    w2s = w2s.astype(jnp.bfloat16)
    w3s = jnp.transpose(conv3_w, (1, 0, 2, 3)).reshape(3, 192, 128)
    w3s = w3s.astype(jnp.bfloat16)

    flat = pl.pallas_call(
        _conv_stack_kernel,
        out_shape=jax.ShapeDtypeStruct((N, 18, 18 * 128), jnp.bfloat16),
        grid=(N // NB,),
        in_specs=[
            pl.BlockSpec((NB, 73, 146), lambda i: (i, 0, 0)),
            pl.BlockSpec((NB, 73, 146), lambda i: (i, 0, 0)),
            pl.BlockSpec((2, 3, 146, 2304), lambda i: (0, 0, 0, 0)),
            pl.BlockSpec((1, 2304), lambda i: (0, 0)),
            pl.BlockSpec((3, 96, 64), lambda i: (0, 0, 0)),
            pl.BlockSpec((1, 64), lambda i: (0, 0)),
            pl.BlockSpec((3, 192, 128), lambda i: (0, 0, 0)),
            pl.BlockSpec((1, 128), lambda i: (0, 0)),
        ],
        out_specs=pl.BlockSpec((NB, 18, 18 * 128), lambda i: (i, 0, 0)),
        compiler_params=pltpu.CompilerParams(
            dimension_semantics=("parallel",)),
    )(xe, xo, band, bias1, w2s, conv2_b.reshape(1, 64), w3s,
      conv3_b.reshape(1, 128))

    a_flat = flat.reshape(N, 41472)

    tk = 41472 // 4
    partials = pl.pallas_call(
        _fc_kernel,
        out_shape=jax.ShapeDtypeStruct((2, N, 3), jnp.float32),
        grid=(2, 4),
        in_specs=[
            pl.BlockSpec((N, tk), lambda h, k: (0, k)),
            pl.BlockSpec((tk, 128), lambda h, k: (k, h)),
            pl.BlockSpec((1, 128), lambda h, k: (0, h)),
            pl.BlockSpec((128, 3), lambda h, k: (h, 0)),
        ],
        out_specs=pl.BlockSpec((1, N, 3), lambda h, k: (h, 0, 0)),
        scratch_shapes=[pltpu.VMEM((N, 128), jnp.float32)],
        compiler_params=pltpu.CompilerParams(
            dimension_semantics=("parallel", "arbitrary")),
    )(a_flat, fc1_w, fc1_b.reshape(1, 256).astype(jnp.float32), fc2_w)

    return jnp.sum(partials, axis=0) + fc2_b.reshape(1, 3)


# speedup vs baseline: 1.2709x; 1.0495x over previous
import functools

import jax
import jax.numpy as jnp
from jax.experimental import pallas as pl
from jax.experimental.pallas import tpu as pltpu


NB = 2


def _conv_stack_kernel(xp_ref, b1_ref, bias1_ref, w2_ref, b2_ref,
                       w3_ref, b3_ref, o_ref):
    f32 = jnp.float32
    bf16 = jnp.bfloat16

    x = xp_ref[...].reshape(NB, 73, 2, 146)
    xe = x[:, :, 0, :]
    xo = x[:, :, 1, :]
    acc1 = [[None, None], [None, None]]
    for hp in range(2):
        for dy in range(3):
            s = hp + dy
            src = xe if s % 2 == 0 else xo
            lhs = src[:, s // 2:s // 2 + 72, :].reshape(NB * 72, 146)
            for wp in range(2):
                d = jnp.dot(lhs, b1_ref[wp, dy], preferred_element_type=f32)
                acc1[hp][wp] = d if acc1[hp][wp] is None else acc1[hp][wp] + d
    m = jnp.maximum(jnp.maximum(acc1[0][0], acc1[0][1]),
                    jnp.maximum(acc1[1][0], acc1[1][1]))
    y = jnp.maximum(m + bias1_ref[0], 0.0).astype(bf16)
    y = y.reshape(NB, 72, 72, 32)

    zc = jnp.zeros((NB, 72, 1, 32), bf16)
    zr = jnp.zeros((NB, 1, 74, 32), bf16)
    p = jnp.concatenate([zc, y, zc], axis=2)
    p = jnp.concatenate([zr, p, zr], axis=1)
    st = jnp.concatenate([p[:, 0:72], p[:, 1:73], p[:, 2:74]], axis=3)
    stv = st.reshape(NB, 36, 2, 37, 2, 96)
    acc2 = [[None, None], [None, None]]
    for hp in range(2):
        for wp in range(2):
            for dx in range(3):
                w0 = wp + dx
                patch = stv[:, :, hp, w0 // 2:w0 // 2 + 36, w0 % 2, :]
                patch = patch.reshape(NB * 36 * 36, 96)
                d = jnp.dot(patch, w2_ref[dx], preferred_element_type=f32)
                acc2[hp][wp] = d if acc2[hp][wp] is None else acc2[hp][wp] + d
    m = jnp.maximum(jnp.maximum(acc2[0][0], acc2[0][1]),
                    jnp.maximum(acc2[1][0], acc2[1][1]))
    y = jnp.maximum(m + b2_ref[0], 0.0).astype(bf16)
    y = y.reshape(NB, 36, 36, 64)

    zc = jnp.zeros((NB, 36, 1, 64), bf16)
    zr = jnp.zeros((NB, 1, 38, 64), bf16)
    p = jnp.concatenate([zc, y, zc], axis=2)
    p = jnp.concatenate([zr, p, zr], axis=1)
    st = jnp.concatenate([p[:, 0:36], p[:, 1:37], p[:, 2:38]], axis=3)
    stv = st.reshape(NB, 18, 2, 19, 2, 192)
    acc3 = [[None, None], [None, None]]
    for hp in range(2):
        for wp in range(2):
            for dx in range(3):
                w0 = wp + dx
                patch = stv[:, :, hp, w0 // 2:w0 // 2 + 18, w0 % 2, :]
                patch = patch.reshape(NB * 18 * 18, 192)
                d = jnp.dot(patch, w3_ref[dx], preferred_element_type=f32)
                acc3[hp][wp] = d if acc3[hp][wp] is None else acc3[hp][wp] + d
    m = jnp.maximum(jnp.maximum(acc3[0][0], acc3[0][1]),
                    jnp.maximum(acc3[1][0], acc3[1][1]))
    y = jnp.maximum(m + b3_ref[0], 0.0).astype(bf16)
    o_ref[...] = y.reshape(NB, 18, 18 * 128)


def _fc_kernel(a_ref, w1_ref, b1_ref, w2_ref, o_ref, acc_ref):
    k = pl.program_id(1)

    @pl.when(k == 0)
    def _():
        acc_ref[...] = jnp.zeros_like(acc_ref)

    acc_ref[...] += jnp.dot(a_ref[...], w1_ref[...],
                            preferred_element_type=jnp.float32)

    @pl.when(k == pl.num_programs(1) - 1)
    def _():
        h = jnp.maximum(acc_ref[...] + b1_ref[...], 0.0).astype(jnp.bfloat16)
        o_ref[0] = jnp.dot(h, w2_ref[...],
                           preferred_element_type=jnp.float32)


def kernel(conv1_w, conv1_b, conv2_w, conv2_b, conv3_w, conv3_b,
           fc1_w, fc1_b, fc2_w, fc2_b, x):
    N = x.shape[0]

    xp = jnp.pad(x.reshape(N, 144, 144), ((0, 0), (1, 1), (1, 1)))
    xp = xp.astype(jnp.bfloat16)

    eye = [jnp.eye(146, 144, -dx, dtype=jnp.float32) for dx in range(3)]
    band = sum(eye[dx][None, :, :, None] * conv1_w[:, dx, 0, :][:, None, None, :]
               for dx in range(3))
    band = band.reshape(3, 146, 72, 2, 32)
    band = jnp.moveaxis(band, 3, 0).reshape(2, 3, 146, 72 * 32)
    band = band.astype(jnp.bfloat16)
    bias1 = jnp.tile(conv1_b, 72).reshape(1, 72 * 32)

    w2s = jnp.transpose(conv2_w, (1, 0, 2, 3)).reshape(3, 96, 64)
    w2s = w2s.astype(jnp.bfloat16)
    w3s = jnp.transpose(conv3_w, (1, 0, 2, 3)).reshape(3, 192, 128)
    w3s = w3s.astype(jnp.bfloat16)

    flat = pl.pallas_call(
        _conv_stack_kernel,
        out_shape=jax.ShapeDtypeStruct((N, 18, 18 * 128), jnp.bfloat16),
        grid=(N // NB,),
        in_specs=[
            pl.BlockSpec((NB, 146, 146), lambda i: (i, 0, 0)),
            pl.BlockSpec((2, 3, 146, 2304), lambda i: (0, 0, 0, 0)),
            pl.BlockSpec((1, 2304), lambda i: (0, 0)),
            pl.BlockSpec((3, 96, 64), lambda i: (0, 0, 0)),
            pl.BlockSpec((1, 64), lambda i: (0, 0)),
            pl.BlockSpec((3, 192, 128), lambda i: (0, 0, 0)),
            pl.BlockSpec((1, 128), lambda i: (0, 0)),
        ],
        out_specs=pl.BlockSpec((NB, 18, 18 * 128), lambda i: (i, 0, 0)),
        compiler_params=pltpu.CompilerParams(
            dimension_semantics=("parallel",)),
    )(xp, band, bias1, w2s, conv2_b.reshape(1, 64), w3s,
      conv3_b.reshape(1, 128))

    a_flat = flat.reshape(N, 41472)

    tk = 41472 // 4
    partials = pl.pallas_call(
        _fc_kernel,
        out_shape=jax.ShapeDtypeStruct((2, N, 3), jnp.float32),
        grid=(2, 4),
        in_specs=[
            pl.BlockSpec((N, tk), lambda h, k: (0, k)),
            pl.BlockSpec((tk, 128), lambda h, k: (k, h)),
            pl.BlockSpec((1, 128), lambda h, k: (0, h)),
            pl.BlockSpec((128, 3), lambda h, k: (h, 0)),
        ],
        out_specs=pl.BlockSpec((1, N, 3), lambda h, k: (h, 0, 0)),
        scratch_shapes=[pltpu.VMEM((N, 128), jnp.float32)],
        compiler_params=pltpu.CompilerParams(
            dimension_semantics=("parallel", "arbitrary")),
    )(a_flat, fc1_w, fc1_b.reshape(1, 256).astype(jnp.float32), fc2_w)

    return jnp.sum(partials, axis=0) + fc2_b.reshape(1, 3)
